# baseline (speedup 1.0000x reference)
import functools

import jax
import jax.numpy as jnp
from jax import lax
from jax.experimental import pallas as pl
from jax.experimental.pallas import tpu as pltpu
from jax.experimental.pallas import tpu_sc as plsc

N, E, D, G = 10000, 320000, 128, 64
NC, NS = 2, 16
NW = NC * NS
CHUNK = 120
RB = 3
RI = 2 * RB
NCH = 114
A0, A1 = 108, 60
EPAD = NS * (A0 + A1) * CHUNK
NPAD = 10112
RPT = NPAD // NS

ROWS = 632
NBLK = NPAD // ROWS


def _sc_aggregate():
    mesh = plsc.VectorSubcoreMesh(core_axis_name="c", subcore_axis_name="s")

    @functools.partial(
        pl.kernel,
        mesh=mesh,
        out_type=jax.ShapeDtypeStruct((NC, NPAD, D), jnp.float32),
        scratch_types=(
            [pltpu.VMEM((CHUNK,), jnp.int32)] * RI +
            [pltpu.VMEM((CHUNK,), jnp.int32)] * RB +
            [pltpu.VMEM((CHUNK, D), jnp.float32)] * RB +
            [pltpu.VMEM_SHARED((NPAD, D), jnp.float32)] +
            [pltpu.SemaphoreType.DMA] * (RI + 2 * RB)
        ),
    )
    def agg(src_hbm, dst_hbm, feat_hbm, out_hbm, *refs):
        sb = refs[0:RI]
        db = refs[RI:RI + RB]
        rows = refs[RI + RB:RI + 2 * RB]
        acc = refs[RI + 2 * RB]
        isem = refs[RI + 2 * RB + 1:2 * RI + 2 * RB + 1]
        dsem = refs[2 * RI + 2 * RB + 1:2 * RI + 3 * RB + 1]
        gsem = refs[2 * RI + 3 * RB + 1:2 * RI + 4 * RB + 1]
        c = lax.axis_index("c")
        s = lax.axis_index("s")
        wid = c * NS + s
        active = jnp.where(c == 0, A0, A1)

        for t in range(RI):
            pltpu.async_copy(src_hbm.at[wid, t], sb[t], isem[t])
        for t in range(RB):
            pltpu.async_copy(dst_hbm.at[wid, t], db[t], dsem[t])
        pltpu.sync_copy(feat_hbm.at[pl.ds(s * RPT, RPT)], acc.at[pl.ds(s * RPT, RPT)])
        for t in range(RB):
            pltpu.make_async_copy(src_hbm.at[wid, t], sb[t], isem[t]).wait()
            pltpu.async_copy(feat_hbm.at[sb[t]], rows[t], gsem[t])
        plsc.subcore_barrier()

        def slot(k, ti, hasI, hasB):
            tb = ti % RB

            @pl.when(k < active)
            def _():
                pltpu.make_async_copy(feat_hbm.at[sb[ti]], rows[tb], gsem[tb]).wait()
                pltpu.make_async_copy(dst_hbm.at[wid, k], db[tb], dsem[tb]).wait()
                pltpu.sync_copy(rows[tb], acc.at[db[tb]], add=True)

            if hasI:
                @pl.when(k + RI < active)
                def _():
                    pltpu.async_copy(src_hbm.at[wid, k + RI], sb[ti], isem[ti])
            if hasB:
                @pl.when(k + RB < active)
                def _():
                    ui = (ti + RB) % RI
                    pltpu.async_copy(dst_hbm.at[wid, k + RB], db[tb], dsem[tb])
                    pltpu.make_async_copy(src_hbm.at[wid, k + RB], sb[ui], isem[ui]).wait()
                    pltpu.async_copy(feat_hbm.at[sb[ui]], rows[tb], gsem[tb])

        def body(j, carry):
            for ti in range(RI):
                slot(RI * j + ti, ti, True, True)
            return carry

        lax.fori_loop(0, NCH // RI - 1, body, 0)
        for ti in range(RI):
            slot(NCH - RI + ti, ti, False, ti < RI - RB)

        plsc.subcore_barrier()
        pltpu.sync_copy(acc.at[pl.ds(s * RPT, RPT)], out_hbm.at[c, pl.ds(s * RPT, RPT)])

    return agg


def _tc_mlp(parts, feat, wa, ba, wb, bb):
    def body(pp, xr, wa_r, ba_r, wb_r, bb_r, out):
        h0 = pp[0] + pp[1] - xr[...]
        t = jnp.dot(h0, wa_r[...], preferred_element_type=jnp.float32) + ba_r[...]
        t = jnp.maximum(t, 0.0)
        o = jnp.dot(t, wb_r[...], preferred_element_type=jnp.float32) + bb_r[...]
        out[...] = jnp.maximum(o, 0.0)

    return pl.pallas_call(
        body,
        grid=(NBLK,),
        in_specs=[
            pl.BlockSpec((NC, ROWS, D), lambda i: (0, i, 0)),
            pl.BlockSpec((ROWS, D), lambda i: (i, 0)),
            pl.BlockSpec((D, D), lambda i: (0, 0)),
            pl.BlockSpec((1, D), lambda i: (0, 0)),
            pl.BlockSpec((D, D), lambda i: (0, 0)),
            pl.BlockSpec((1, D), lambda i: (0, 0)),
        ],
        out_specs=pl.BlockSpec((ROWS, D), lambda i: (i, 0)),
        out_shape=jax.ShapeDtypeStruct((NPAD, D), jnp.float32),
    )(parts, feat, wa, ba, wb, bb)


def _tc_conv2_pool_head(parts, feat, batch3, wa, ba, wb, bb, wl1, bl1, wl2p, bl2p):
    def body(pp, xr, b_r, wa_r, ba_r, wb_r, bb_r, wl1_r, bl1_r, wl2_r, bl2_r,
             out, acc_s, acc_c):
        i = pl.program_id(0)
        h0 = pp[0] + pp[1] - xr[...]
        t = jnp.dot(h0, wa_r[...], preferred_element_type=jnp.float32) + ba_r[...]
        t = jnp.maximum(t, 0.0)
        h = jnp.dot(t, wb_r[...], preferred_element_type=jnp.float32) + bb_r[...]
        h = jnp.maximum(h, 0.0)
        ids = b_r[0]
        oh = (lax.broadcasted_iota(jnp.int32, (G, ROWS), 0) == ids).astype(jnp.float32)
        ps = jnp.dot(oh, h, preferred_element_type=jnp.float32)
        pc = jnp.broadcast_to(jnp.sum(oh, axis=1, keepdims=True), (G, D))

        @pl.when(i == 0)
        def _():
            acc_s[...] = ps
            acc_c[...] = pc

        @pl.when(i > 0)
        def _():
            acc_s[...] += ps
            acc_c[...] += pc

        @pl.when(i == NBLK - 1)
        def _():
            pooled = acc_s[...] / jnp.maximum(acc_c[...], 1.0)
            o1 = jnp.dot(pooled, wl1_r[...], preferred_element_type=jnp.float32) + bl1_r[...]
            o1 = jnp.maximum(o1, 0.0)
            out[...] = jnp.dot(o1, wl2_r[...], preferred_element_type=jnp.float32) + bl2_r[...]

    return pl.pallas_call(
        body,
        grid=(NBLK,),
        in_specs=[
            pl.BlockSpec((NC, ROWS, D), lambda i: (0, i, 0)),
            pl.BlockSpec((ROWS, D), lambda i: (i, 0)),
            pl.BlockSpec((1, 1, ROWS), lambda i: (i, 0, 0)),
            pl.BlockSpec((D, D), lambda i: (0, 0)),
            pl.BlockSpec((1, D), lambda i: (0, 0)),
            pl.BlockSpec((D, D), lambda i: (0, 0)),
            pl.BlockSpec((1, D), lambda i: (0, 0)),
            pl.BlockSpec((D, D), lambda i: (0, 0)),
            pl.BlockSpec((1, D), lambda i: (0, 0)),
            pl.BlockSpec((D, D), lambda i: (0, 0)),
            pl.BlockSpec((1, D), lambda i: (0, 0)),
        ],
        out_specs=pl.BlockSpec((G, D), lambda i: (0, 0)),
        out_shape=jax.ShapeDtypeStruct((G, D), jnp.float32),
        scratch_shapes=[
            pltpu.VMEM((G, D), jnp.float32),
            pltpu.VMEM((G, D), jnp.float32),
        ],
    )(parts, feat, batch3, wa, ba, wb, bb, wl1, bl1, wl2p, bl2p)


def kernel(x, edge_index, batch,
           W1a, b1a, W1b, b1b,
           W2a, b2a, W2b, b2b,
           Wl1, bl1, Wl2, bl2):
    pad = EPAD - E
    src = jnp.concatenate([edge_index[0], jnp.zeros((pad,), jnp.int32)])
    pad_dst = N + (jnp.arange(pad, dtype=jnp.int32) % (NPAD - N))
    dst = jnp.concatenate([edge_index[1], pad_dst])

    def _pack(a):
        cut = NS * A0 * CHUNK
        a0 = jnp.pad(a[:cut].reshape(NS, A0, CHUNK), ((0, 0), (0, NCH - A0), (0, 0)))
        a1 = jnp.pad(a[cut:].reshape(NS, A1, CHUNK), ((0, 0), (0, NCH - A1), (0, 0)))
        return jnp.concatenate([a0, a1], axis=0)

    src = _pack(src)
    dst = _pack(dst)
    xp = jnp.pad(x, ((0, NPAD - N), (0, 0)))
    batch3 = jnp.pad(batch, (0, NPAD - N), constant_values=G).reshape(NBLK, 1, ROWS)
    b1a_r = b1a.reshape(1, D)
    b1b_r = b1b.reshape(1, D)
    b2a_r = b2a.reshape(1, D)
    b2b_r = b2b.reshape(1, D)
    bl1_r = bl1.reshape(1, D)
    O = Wl2.shape[1]
    wl2p = jnp.pad(Wl2, ((0, 0), (0, D - O)))
    bl2p = jnp.pad(bl2, (0, D - O)).reshape(1, D)

    agg = _sc_aggregate()
    parts1 = agg(src, dst, xp)
    h1 = _tc_mlp(parts1, xp, W1a, b1a_r, W1b, b1b_r)
    parts2 = agg(src, dst, h1)
    out128 = _tc_conv2_pool_head(parts2, h1, batch3, W2a, b2a_r, W2b, b2b_r,
                                 Wl1, bl1_r, wl2p, bl2p)
    return out128[:, :O]


# speedup vs baseline: 1.0430x; 1.0430x over previous
import functools

import jax
import jax.numpy as jnp
from jax import lax
from jax.experimental import pallas as pl
from jax.experimental.pallas import tpu as pltpu
from jax.experimental.pallas import tpu_sc as plsc

N, E, D, G = 10000, 320000, 128, 64
NC, NS = 2, 16
NW = NC * NS
CHUNK = 120
RB = 3
RI = 2 * RB
NCH = 114
A0, A1 = 96, 72
EPAD = NS * (A0 + A1) * CHUNK
NPAD = 10112
RPT = NPAD // NS

ROWS = 632
NBLK = NPAD // ROWS


def _sc_aggregate():
    mesh = plsc.VectorSubcoreMesh(core_axis_name="c", subcore_axis_name="s")

    @functools.partial(
        pl.kernel,
        mesh=mesh,
        out_type=jax.ShapeDtypeStruct((NC, NPAD, D), jnp.float32),
        scratch_types=(
            [pltpu.VMEM((CHUNK,), jnp.int32)] * RI +
            [pltpu.VMEM((CHUNK,), jnp.int32)] * RB +
            [pltpu.VMEM((CHUNK, D), jnp.float32)] * RB +
            [pltpu.VMEM_SHARED((NPAD, D), jnp.float32)] +
            [pltpu.SemaphoreType.DMA] * (RI + 2 * RB)
        ),
    )
    def agg(src_hbm, dst_hbm, feat_hbm, out_hbm, *refs):
        sb = refs[0:RI]
        db = refs[RI:RI + RB]
        rows = refs[RI + RB:RI + 2 * RB]
        acc = refs[RI + 2 * RB]
        isem = refs[RI + 2 * RB + 1:2 * RI + 2 * RB + 1]
        dsem = refs[2 * RI + 2 * RB + 1:2 * RI + 3 * RB + 1]
        gsem = refs[2 * RI + 3 * RB + 1:2 * RI + 4 * RB + 1]
        c = lax.axis_index("c")
        s = lax.axis_index("s")
        wid = c * NS + s
        active = jnp.where(c == 0, A0, A1)

        for t in range(RI):
            pltpu.async_copy(src_hbm.at[wid, t], sb[t], isem[t])
        for t in range(RB):
            pltpu.async_copy(dst_hbm.at[wid, t], db[t], dsem[t])
        pltpu.sync_copy(feat_hbm.at[pl.ds(s * RPT, RPT)], acc.at[pl.ds(s * RPT, RPT)])
        for t in range(RB):
            pltpu.make_async_copy(src_hbm.at[wid, t], sb[t], isem[t]).wait()
            pltpu.async_copy(feat_hbm.at[sb[t]], rows[t], gsem[t])
        plsc.subcore_barrier()

        def slot(k, ti, hasI, hasB):
            tb = ti % RB

            @pl.when(k < active)
            def _():
                pltpu.make_async_copy(feat_hbm.at[sb[ti]], rows[tb], gsem[tb]).wait()
                pltpu.make_async_copy(dst_hbm.at[wid, k], db[tb], dsem[tb]).wait()
                pltpu.sync_copy(rows[tb], acc.at[db[tb]], add=True)

            if hasI:
                @pl.when(k + RI < active)
                def _():
                    pltpu.async_copy(src_hbm.at[wid, k + RI], sb[ti], isem[ti])
            if hasB:
                @pl.when(k + RB < active)
                def _():
                    ui = (ti + RB) % RI
                    pltpu.async_copy(dst_hbm.at[wid, k + RB], db[tb], dsem[tb])
                    pltpu.make_async_copy(src_hbm.at[wid, k + RB], sb[ui], isem[ui]).wait()
                    pltpu.async_copy(feat_hbm.at[sb[ui]], rows[tb], gsem[tb])

        def body(j, carry):
            for ti in range(RI):
                slot(RI * j + ti, ti, True, True)
            return carry

        lax.fori_loop(0, NCH // RI - 1, body, 0)
        for ti in range(RI):
            slot(NCH - RI + ti, ti, False, ti < RI - RB)

        plsc.subcore_barrier()
        pltpu.sync_copy(acc.at[pl.ds(s * RPT, RPT)], out_hbm.at[c, pl.ds(s * RPT, RPT)])

    return agg


def _tc_mlp(parts, feat, wa, ba, wb, bb):
    def body(pp, xr, wa_r, ba_r, wb_r, bb_r, out):
        h0 = pp[0] + pp[1] - xr[...]
        t = jnp.dot(h0, wa_r[...], preferred_element_type=jnp.float32) + ba_r[...]
        t = jnp.maximum(t, 0.0)
        o = jnp.dot(t, wb_r[...], preferred_element_type=jnp.float32) + bb_r[...]
        out[...] = jnp.maximum(o, 0.0)

    return pl.pallas_call(
        body,
        grid=(NBLK,),
        in_specs=[
            pl.BlockSpec((NC, ROWS, D), lambda i: (0, i, 0)),
            pl.BlockSpec((ROWS, D), lambda i: (i, 0)),
            pl.BlockSpec((D, D), lambda i: (0, 0)),
            pl.BlockSpec((1, D), lambda i: (0, 0)),
            pl.BlockSpec((D, D), lambda i: (0, 0)),
            pl.BlockSpec((1, D), lambda i: (0, 0)),
        ],
        out_specs=pl.BlockSpec((ROWS, D), lambda i: (i, 0)),
        out_shape=jax.ShapeDtypeStruct((NPAD, D), jnp.float32),
    )(parts, feat, wa, ba, wb, bb)


def _tc_conv2_pool_head(parts, feat, batch3, wa, ba, wb, bb, wl1, bl1, wl2p, bl2p):
    def body(pp, xr, b_r, wa_r, ba_r, wb_r, bb_r, wl1_r, bl1_r, wl2_r, bl2_r,
             out, acc_s, acc_c):
        i = pl.program_id(0)
        h0 = pp[0] + pp[1] - xr[...]
        t = jnp.dot(h0, wa_r[...], preferred_element_type=jnp.float32) + ba_r[...]
        t = jnp.maximum(t, 0.0)
        h = jnp.dot(t, wb_r[...], preferred_element_type=jnp.float32) + bb_r[...]
        h = jnp.maximum(h, 0.0)
        ids = b_r[0]
        oh = (lax.broadcasted_iota(jnp.int32, (G, ROWS), 0) == ids).astype(jnp.float32)
        ps = jnp.dot(oh, h, preferred_element_type=jnp.float32)
        pc = jnp.broadcast_to(jnp.sum(oh, axis=1, keepdims=True), (G, D))

        @pl.when(i == 0)
        def _():
            acc_s[...] = ps
            acc_c[...] = pc

        @pl.when(i > 0)
        def _():
            acc_s[...] += ps
            acc_c[...] += pc

        @pl.when(i == NBLK - 1)
        def _():
            pooled = acc_s[...] / jnp.maximum(acc_c[...], 1.0)
            o1 = jnp.dot(pooled, wl1_r[...], preferred_element_type=jnp.float32) + bl1_r[...]
            o1 = jnp.maximum(o1, 0.0)
            out[...] = jnp.dot(o1, wl2_r[...], preferred_element_type=jnp.float32) + bl2_r[...]

    return pl.pallas_call(
        body,
        grid=(NBLK,),
        in_specs=[
            pl.BlockSpec((NC, ROWS, D), lambda i: (0, i, 0)),
            pl.BlockSpec((ROWS, D), lambda i: (i, 0)),
            pl.BlockSpec((1, 1, ROWS), lambda i: (i, 0, 0)),
            pl.BlockSpec((D, D), lambda i: (0, 0)),
            pl.BlockSpec((1, D), lambda i: (0, 0)),
            pl.BlockSpec((D, D), lambda i: (0, 0)),
            pl.BlockSpec((1, D), lambda i: (0, 0)),
            pl.BlockSpec((D, D), lambda i: (0, 0)),
            pl.BlockSpec((1, D), lambda i: (0, 0)),
            pl.BlockSpec((D, D), lambda i: (0, 0)),
            pl.BlockSpec((1, D), lambda i: (0, 0)),
        ],
        out_specs=pl.BlockSpec((G, D), lambda i: (0, 0)),
        out_shape=jax.ShapeDtypeStruct((G, D), jnp.float32),
        scratch_shapes=[
            pltpu.VMEM((G, D), jnp.float32),
            pltpu.VMEM((G, D), jnp.float32),
        ],
    )(parts, feat, batch3, wa, ba, wb, bb, wl1, bl1, wl2p, bl2p)


def kernel(x, edge_index, batch,
           W1a, b1a, W1b, b1b,
           W2a, b2a, W2b, b2b,
           Wl1, bl1, Wl2, bl2):
    pad = EPAD - E
    src = jnp.concatenate([edge_index[0], jnp.zeros((pad,), jnp.int32)])
    pad_dst = N + (jnp.arange(pad, dtype=jnp.int32) % (NPAD - N))
    dst = jnp.concatenate([edge_index[1], pad_dst])

    def _pack(a):
        cut = NS * A0 * CHUNK
        a0 = jnp.pad(a[:cut].reshape(NS, A0, CHUNK), ((0, 0), (0, NCH - A0), (0, 0)))
        a1 = jnp.pad(a[cut:].reshape(NS, A1, CHUNK), ((0, 0), (0, NCH - A1), (0, 0)))
        return jnp.concatenate([a0, a1], axis=0)

    src = _pack(src)
    dst = _pack(dst)
    xp = jnp.pad(x, ((0, NPAD - N), (0, 0)))
    batch3 = jnp.pad(batch, (0, NPAD - N), constant_values=G).reshape(NBLK, 1, ROWS)
    b1a_r = b1a.reshape(1, D)
    b1b_r = b1b.reshape(1, D)
    b2a_r = b2a.reshape(1, D)
    b2b_r = b2b.reshape(1, D)
    bl1_r = bl1.reshape(1, D)
    O = Wl2.shape[1]
    wl2p = jnp.pad(Wl2, ((0, 0), (0, D - O)))
    bl2p = jnp.pad(bl2, (0, D - O)).reshape(1, D)

    agg = _sc_aggregate()
    parts1 = agg(src, dst, xp)
    h1 = _tc_mlp(parts1, xp, W1a, b1a_r, W1b, b1b_r)
    parts2 = agg(src, dst, h1)
    out128 = _tc_conv2_pool_head(parts2, h1, batch3, W2a, b2a_r, W2b, b2b_r,
                                 Wl1, bl1_r, wl2p, bl2p)
    return out128[:, :O]
